# baseline (speedup 1.0000x reference)
import jax
import jax.numpy as jnp
from jax.experimental import pallas as pl
from jax.experimental.pallas import tpu as pltpu

B, T, H, A, E, L = 128, 512, 1024, 128, 16, 1000


def _adapter_head_kernel(lh_ref, vids_ref, Wd_ref, bd_ref, Wu_ref, bu_ref,
                         Wc_ref, bc_ref, out_ref):
    x = lh_ref[...]
    vids = vids_ref[...]
    up = jnp.zeros((B, H), dtype=jnp.float32)
    for e in range(E):
        m = (vids == e).astype(jnp.float32)
        h = jnp.dot(x, Wd_ref[e], preferred_element_type=jnp.float32)
        h = jnp.maximum(h + bd_ref[e], 0.0) * m
        up = up + jnp.dot(h, Wu_ref[e], preferred_element_type=jnp.float32)
        up = up + m * bu_ref[e]
    out = x + up
    logits = jnp.dot(out, Wc_ref[...], preferred_element_type=jnp.float32)
    out_ref[...] = logits + bc_ref[...]


def kernel(last_hidden, attention_mask, variety_ids, W_down, b_down, W_up,
           b_up, W_c, b_c):
    vids = variety_ids.reshape(B, 1)
    lh_flat = last_hidden.reshape(B, T * H)
    logits = pl.pallas_call(
        _adapter_head_kernel,
        grid=(1,),
        in_specs=[
            pl.BlockSpec((B, H), lambda i: (0, 0)),
            pl.BlockSpec((B, 1), lambda i: (0, 0)),
            pl.BlockSpec((E, H, A), lambda i: (0, 0, 0)),
            pl.BlockSpec((E, 1, A), lambda i: (0, 0, 0)),
            pl.BlockSpec((E, A, H), lambda i: (0, 0, 0)),
            pl.BlockSpec((E, 1, H), lambda i: (0, 0, 0)),
            pl.BlockSpec((H, L), lambda i: (0, 0)),
            pl.BlockSpec((1, L), lambda i: (0, 0)),
        ],
        out_specs=pl.BlockSpec((B, L), lambda i: (0, 0)),
        out_shape=jax.ShapeDtypeStruct((B, L), jnp.float32),
    )(lh_flat, vids, W_down, b_down.reshape(E, 1, A), W_up,
      b_up.reshape(E, 1, H), W_c, b_c.reshape(1, L))
    return logits


# speedup vs baseline: 8.4979x; 8.4979x over previous
import jax
import jax.numpy as jnp
from jax.experimental import pallas as pl
from jax.experimental.pallas import tpu as pltpu

B, T, H, A, E, L = 128, 512, 1024, 128, 16, 1000
GE = 4
NG = E // GE
KC = 256
NK = H // KC


def _adapter_head_kernel(x_ref, vids_ref, Wd_ref, bd_ref, Wu_ref, bu_ref,
                         Wc_ref, bc_ref, out_ref, act_ref):
    i = pl.program_id(0)
    x = x_ref[...]
    vids = vids_ref[...]

    @pl.when(i == 0)
    def _init():
        iota = jax.lax.broadcasted_iota(jnp.int32, (B, E), 1)
        onehot = (vids == iota).astype(jnp.float32)
        act_ref[...] = x + jnp.dot(onehot, bu_ref[...],
                                   preferred_element_type=jnp.float32)

    @pl.when(i < NG)
    def _adapters():
        iota = jax.lax.broadcasted_iota(jnp.int32, (B, E), 1)
        onehot = (vids == iota).astype(jnp.float32)
        bdg = jnp.dot(onehot, bd_ref[...],
                      preferred_element_type=jnp.float32)
        up = jnp.zeros((B, H), dtype=jnp.float32)
        for j in range(GE):
            e = i * GE + j
            m = (vids == e).astype(jnp.float32)
            h = jnp.dot(x, Wd_ref[j], preferred_element_type=jnp.float32)
            h = jnp.maximum(h + bdg, 0.0) * m
            up = up + jnp.dot(h, Wu_ref[j], preferred_element_type=jnp.float32)
        act_ref[...] += up

    @pl.when(i >= NG)
    def _classifier():
        k = i - NG
        part = jnp.dot(act_ref[:, pl.ds(k * KC, KC)], Wc_ref[...],
                       preferred_element_type=jnp.float32)

        @pl.when(i == NG)
        def _first():
            out_ref[...] = part + bc_ref[...]

        @pl.when(i > NG)
        def _rest():
            out_ref[...] += part


def kernel(last_hidden, attention_mask, variety_ids, W_down, b_down, W_up,
           b_up, W_c, b_c):
    x = last_hidden[:, 0, :]
    vids = variety_ids.reshape(B, 1)
    logits = pl.pallas_call(
        _adapter_head_kernel,
        grid=(NG + NK,),
        in_specs=[
            pl.BlockSpec((B, H), lambda i: (0, 0)),
            pl.BlockSpec((B, 1), lambda i: (0, 0)),
            pl.BlockSpec((GE, H, A),
                         lambda i: (jnp.minimum(i, NG - 1), 0, 0)),
            pl.BlockSpec((E, A), lambda i: (0, 0)),
            pl.BlockSpec((GE, A, H),
                         lambda i: (jnp.minimum(i, NG - 1), 0, 0)),
            pl.BlockSpec((E, H), lambda i: (0, 0)),
            pl.BlockSpec((KC, L),
                         lambda i: (jnp.maximum(i - NG, 0), 0)),
            pl.BlockSpec((1, L), lambda i: (0, 0)),
        ],
        out_specs=pl.BlockSpec((B, L), lambda i: (0, 0)),
        out_shape=jax.ShapeDtypeStruct((B, L), jnp.float32),
        scratch_shapes=[pltpu.VMEM((B, H), jnp.float32)],
        compiler_params=pltpu.CompilerParams(
            dimension_semantics=("arbitrary",),
        ),
    )(x, vids, W_down, b_down, W_up, b_up, W_c, b_c.reshape(1, L))
    return logits


# speedup vs baseline: 9.3249x; 1.0973x over previous
import jax
import jax.numpy as jnp
from jax.experimental import pallas as pl
from jax.experimental.pallas import tpu as pltpu

B, T, H, A, E, L = 128, 512, 1024, 128, 16, 1000
GE = 2
NG = E // GE
KC = 128
NK = H // KC


def _adapter_head_kernel(x_ref, vids_ref, bd_ref, bu_ref, bc_ref,
                         Wd_hbm, Wu_hbm, Wc_hbm,
                         out_ref,
                         wd_buf, wu_buf, wc_buf,
                         wd_sem, wu_sem, wc_sem):
    for g in range(NG):
        pltpu.make_async_copy(Wd_hbm.at[pl.ds(g * GE, GE)],
                              wd_buf.at[g], wd_sem.at[g]).start()
        pltpu.make_async_copy(Wu_hbm.at[pl.ds(g * GE, GE)],
                              wu_buf.at[g], wu_sem.at[g]).start()
    for k in range(NK):
        pltpu.make_async_copy(Wc_hbm.at[pl.ds(k * KC, KC)],
                              wc_buf.at[k], wc_sem.at[k]).start()

    x = x_ref[...]
    vids = vids_ref[...]
    iota = jax.lax.broadcasted_iota(jnp.int32, (B, E), 1)
    onehot = (vids == iota).astype(jnp.float32)
    bdg = jnp.dot(onehot, bd_ref[...],
                  preferred_element_type=jnp.float32)
    act = x + jnp.dot(onehot, bu_ref[...],
                      preferred_element_type=jnp.float32)
    for g in range(NG):
        pltpu.make_async_copy(Wd_hbm.at[pl.ds(g * GE, GE)],
                              wd_buf.at[g], wd_sem.at[g]).wait()
        pltpu.make_async_copy(Wu_hbm.at[pl.ds(g * GE, GE)],
                              wu_buf.at[g], wu_sem.at[g]).wait()
        for j in range(GE):
            e = g * GE + j
            m = (vids == e).astype(jnp.float32)
            h = jnp.dot(x, wd_buf[g, j], preferred_element_type=jnp.float32)
            h = jnp.maximum(h + bdg, 0.0) * m
            act = act + jnp.dot(h, wu_buf[g, j],
                                preferred_element_type=jnp.float32)

    acc = jnp.broadcast_to(bc_ref[...], (B, L))
    for k in range(NK):
        pltpu.make_async_copy(Wc_hbm.at[pl.ds(k * KC, KC)],
                              wc_buf.at[k], wc_sem.at[k]).wait()
        acc = acc + jnp.dot(act[:, k * KC:(k + 1) * KC], wc_buf[k],
                            preferred_element_type=jnp.float32)
    out_ref[...] = acc


def kernel(last_hidden, attention_mask, variety_ids, W_down, b_down, W_up,
           b_up, W_c, b_c):
    x = last_hidden[:, 0, :]
    vids = variety_ids.reshape(B, 1)
    logits = pl.pallas_call(
        _adapter_head_kernel,
        grid=(1,),
        in_specs=[
            pl.BlockSpec((B, H), lambda i: (0, 0)),
            pl.BlockSpec((B, 1), lambda i: (0, 0)),
            pl.BlockSpec((E, A), lambda i: (0, 0)),
            pl.BlockSpec((E, H), lambda i: (0, 0)),
            pl.BlockSpec((1, L), lambda i: (0, 0)),
            pl.BlockSpec(memory_space=pltpu.MemorySpace.HBM),
            pl.BlockSpec(memory_space=pltpu.MemorySpace.HBM),
            pl.BlockSpec(memory_space=pltpu.MemorySpace.HBM),
        ],
        out_specs=pl.BlockSpec((B, L), lambda i: (0, 0)),
        out_shape=jax.ShapeDtypeStruct((B, L), jnp.float32),
        scratch_shapes=[
            pltpu.VMEM((NG, GE, H, A), jnp.float32),
            pltpu.VMEM((NG, GE, A, H), jnp.float32),
            pltpu.VMEM((NK, KC, L), jnp.float32),
            pltpu.SemaphoreType.DMA((NG,)),
            pltpu.SemaphoreType.DMA((NG,)),
            pltpu.SemaphoreType.DMA((NK,)),
        ],
    )(x, vids, b_down, b_up, b_c.reshape(1, L), W_down, W_up, W_c)
    return logits


# speedup vs baseline: 9.4001x; 1.0081x over previous
import jax
import jax.numpy as jnp
from jax.experimental import pallas as pl
from jax.experimental.pallas import tpu as pltpu

B, T, H, A, E, L = 128, 512, 1024, 128, 16, 1000
GE = 4
NG = E // GE
KC = 256
NK = H // KC


def _adapter_head_kernel(x_ref, vids_ref, bd_ref, bu_ref, bc_ref,
                         Wd_hbm, Wu_hbm, Wc_hbm,
                         out_ref,
                         wd_buf, wu_buf, wc_buf,
                         wd_sem, wu_sem, wc_sem):
    for g in range(NG):
        pltpu.make_async_copy(Wd_hbm.at[pl.ds(g * GE, GE)],
                              wd_buf.at[g], wd_sem.at[g]).start()
        pltpu.make_async_copy(Wu_hbm.at[pl.ds(g * GE, GE)],
                              wu_buf.at[g], wu_sem.at[g]).start()
    for k in range(NK):
        pltpu.make_async_copy(Wc_hbm.at[pl.ds(k * KC, KC)],
                              wc_buf.at[k], wc_sem.at[k]).start()

    x = x_ref[...]
    vids = vids_ref[...]
    iota = jax.lax.broadcasted_iota(jnp.int32, (B, E), 1)
    onehot = (vids == iota).astype(jnp.float32)
    bdg = jnp.dot(onehot, bd_ref[...],
                  preferred_element_type=jnp.float32)
    act = x + jnp.dot(onehot, bu_ref[...],
                      preferred_element_type=jnp.float32)
    for g in range(NG):
        pltpu.make_async_copy(Wd_hbm.at[pl.ds(g * GE, GE)],
                              wd_buf.at[g], wd_sem.at[g]).wait()
        pltpu.make_async_copy(Wu_hbm.at[pl.ds(g * GE, GE)],
                              wu_buf.at[g], wu_sem.at[g]).wait()
        for j in range(GE):
            e = g * GE + j
            m = (vids == e).astype(jnp.float32)
            h = jnp.dot(x, wd_buf[g, j], preferred_element_type=jnp.float32)
            h = jnp.maximum(h + bdg, 0.0) * m
            act = act + jnp.dot(h, wu_buf[g, j],
                                preferred_element_type=jnp.float32)

    acc = jnp.broadcast_to(bc_ref[...], (B, L))
    for k in range(NK):
        pltpu.make_async_copy(Wc_hbm.at[pl.ds(k * KC, KC)],
                              wc_buf.at[k], wc_sem.at[k]).wait()
        acc = acc + jnp.dot(act[:, k * KC:(k + 1) * KC], wc_buf[k],
                            preferred_element_type=jnp.float32)
    out_ref[...] = acc


def kernel(last_hidden, attention_mask, variety_ids, W_down, b_down, W_up,
           b_up, W_c, b_c):
    x = last_hidden[:, 0, :]
    vids = variety_ids.reshape(B, 1)
    logits = pl.pallas_call(
        _adapter_head_kernel,
        grid=(1,),
        in_specs=[
            pl.BlockSpec((B, H), lambda i: (0, 0)),
            pl.BlockSpec((B, 1), lambda i: (0, 0)),
            pl.BlockSpec((E, A), lambda i: (0, 0)),
            pl.BlockSpec((E, H), lambda i: (0, 0)),
            pl.BlockSpec((1, L), lambda i: (0, 0)),
            pl.BlockSpec(memory_space=pltpu.MemorySpace.HBM),
            pl.BlockSpec(memory_space=pltpu.MemorySpace.HBM),
            pl.BlockSpec(memory_space=pltpu.MemorySpace.HBM),
        ],
        out_specs=pl.BlockSpec((B, L), lambda i: (0, 0)),
        out_shape=jax.ShapeDtypeStruct((B, L), jnp.float32),
        scratch_shapes=[
            pltpu.VMEM((NG, GE, H, A), jnp.float32),
            pltpu.VMEM((NG, GE, A, H), jnp.float32),
            pltpu.VMEM((NK, KC, L), jnp.float32),
            pltpu.SemaphoreType.DMA((NG,)),
            pltpu.SemaphoreType.DMA((NG,)),
            pltpu.SemaphoreType.DMA((NK,)),
        ],
    )(x, vids, b_down, b_up, b_c.reshape(1, L), W_down, W_up, W_c)
    return logits
